# baseline (speedup 1.0000x reference)
import jax
import jax.numpy as jnp
from jax import lax
from jax.experimental import pallas as pl
from jax.experimental.pallas import tpu as pltpu
from jax.experimental.pallas import tpu_sc as plsc

N, D, G = 320000, 128, 1024
NCORES, NSUB = 2, 16
NW = NCORES * NSUB
C = N // NW
R = 400
NBLK = C // R
NGRP = R // 16
ACC_W = D + 16


def _sc_body(x_hbm, b_hbm, q_hbm, out_hbm, part_hbm,
             xb0, xb1, b_buf, q_buf, bp_buf, srow, zrow,
             g_st, den_st, acc_st, sem0, sem1):
    lane = lax.iota(jnp.int32, 16)
    zf = jnp.zeros((16,), jnp.float32)
    zi = jnp.zeros((16,), jnp.int32)
    cid = lax.axis_index("c")
    sid = lax.axis_index("s")
    wid = cid * NSUB + sid
    base = wid * C

    def _bcast(vec, idx16):
        dnums = lax.GatherDimensionNumbers(
            offset_dims=(), collapsed_slice_dims=(0,), start_index_map=(0,))
        return lax.gather(vec, idx16[:, None], dnums, (1,),
                          mode=lax.GatherScatterMode.PROMISE_IN_BOUNDS)

    def _allsum(v):
        for shift in (8, 4, 2, 1):
            v = v + _bcast(v, jnp.bitwise_xor(lane, shift))
        return v

    pltpu.sync_copy(b_hbm.at[pl.ds(base, C)], b_buf.at[pl.ds(0, C)])
    pltpu.sync_copy(q_hbm, q_buf)
    bp_off = pl.multiple_of(jnp.maximum(base - 16, 0), 8)
    pltpu.sync_copy(b_hbm.at[pl.ds(bp_off, 16)], bp_buf)

    g_prev = jnp.where(wid == 0, -1, bp_buf[...][15])
    g_first = b_buf[pl.ds(0, 16)][0]
    g_last = b_buf[pl.ds(C - 16, 16)][15]

    for k in range(9):
        zrow[pl.ds(16 * k, 16)] = zf

    def _zero_row(gid):
        pltpu.sync_copy(zrow, out_hbm.at[pl.ds(gid * ACC_W, ACC_W)])

    lax.fori_loop(0, g_first - g_prev,
                  lambda i, c: (_zero_row(g_prev + 1 + i), c)[1], 0)

    @pl.when(wid == NW - 1)
    def _():
        lax.fori_loop(0, G - 1 - g_last,
                      lambda i, c: (_zero_row(g_last + 1 + i), c)[1], 0)

    def _load_accs():
        return [acc_st[pl.ds(16 * k, 16)] for k in range(8)]

    def _emit(g_s, den_vec, accs):
        for k in range(8):
            srow[pl.ds(16 * k, 16)] = accs[k]
        srow[pl.ds(128, 16)] = den_vec

        @pl.when(g_s == g_first)
        def _():
            pltpu.sync_copy(srow, part_hbm.at[pl.ds(wid * ACC_W, ACC_W)])

        @pl.when(g_s != g_first)
        def _():
            pltpu.sync_copy(srow, out_hbm.at[pl.ds(g_s * ACC_W, ACC_W)])

    def _make_group_body(xb, pos0, xb_off, gsz):
        def body(grp, carry):
            s0 = xb_off + grp * gsz
            pos = pos0 + s0
            qk = [q_buf[pl.ds(16 * k, 16)] for k in range(8)]
            g = g_st[...][0]
            uniform = b_buf[pl.ds(pos + gsz - 1, 16)][0] == g

            def _row_e(row):
                xk = [xb[row, pl.ds(16 * k, 16)] for k in range(8)]
                p = xk[0] * qk[0]
                for k in range(1, 8):
                    p = p + xk[k] * qk[k]
                return xk, jnp.exp(_allsum(p))

            @pl.when(uniform)
            def _():
                accs = _load_accs()
                den = den_st[...]
                for r in range(gsz):
                    xk, e_bc = _row_e(s0 + r)
                    accs = [accs[k] + e_bc * xk[k] for k in range(8)]
                    den = den + e_bc
                for k in range(8):
                    acc_st[pl.ds(16 * k, 16)] = accs[k]
                den_st[...] = den

            @pl.when(jnp.logical_not(uniform))
            def _():
                def row_body(r, c):
                    b_r = b_buf[pl.ds(pos + r, 16)][0]
                    g_c = g_st[...][0]
                    pred = b_r != g_c

                    @pl.when(pred)
                    def _():
                        _emit(g_c, den_st[...], _load_accs())
                        lax.fori_loop(0, b_r - g_c - 1,
                                      lambda i, cc:
                                      (_zero_row(g_c + 1 + i), cc)[1], 0)
                        for k in range(8):
                            acc_st[pl.ds(16 * k, 16)] = zf
                        den_st[...] = zf
                        g_st[...] = zi + b_r

                    xk, e_bc = _row_e(s0 + r)
                    for k in range(8):
                        acc_st[pl.ds(16 * k, 16)] = (
                            acc_st[pl.ds(16 * k, 16)] + e_bc * xk[k])
                    den_st[...] = den_st[...] + e_bc
                    return c

                lax.fori_loop(0, gsz, row_body, 0)

            return carry
        return body

    def _process_block(xb, pos0, carry):
        carry = lax.fori_loop(0, R // 32,
                              _make_group_body(xb, pos0, 0, 32), carry)
        return _make_group_body(xb, pos0, (R // 32) * 32, 16)(0, carry)

    g_st[...] = zi + g_first
    den_st[...] = zf
    for k in range(8):
        acc_st[pl.ds(16 * k, 16)] = zf
    carry = 0

    def start(blk, buf, sem):
        pltpu.async_copy(x_hbm.at[pl.ds(base + blk * R, R)], buf, sem)

    def wait(blk, buf, sem):
        pltpu.make_async_copy(
            x_hbm.at[pl.ds(base + blk * R, R)], buf, sem).wait()

    start(0, xb0, sem0)

    def bb_body(bb, carry):
        b_even = 2 * bb
        start(b_even + 1, xb1, sem1)
        wait(b_even, xb0, sem0)
        carry = _process_block(xb0, b_even * R, carry)
        start(b_even + 2, xb0, sem0)
        wait(b_even + 1, xb1, sem1)
        carry = _process_block(xb1, (b_even + 1) * R, carry)
        return carry

    carry = lax.fori_loop(0, (NBLK - 1) // 2, bb_body, carry)
    wait(NBLK - 1, xb0, sem0)
    carry = _process_block(xb0, (NBLK - 1) * R, carry)
    _emit(g_st[...][0], den_st[...], _load_accs())


def _sc_pool(x, b32, query):
    mesh = plsc.VectorSubcoreMesh(core_axis_name="c", subcore_axis_name="s")
    kern = pl.kernel(
        _sc_body,
        mesh=mesh,
        out_type=[jax.ShapeDtypeStruct((G * ACC_W,), jnp.float32),
                  jax.ShapeDtypeStruct((NW * ACC_W,), jnp.float32)],
        scratch_types=[
            pltpu.VMEM((R, D), jnp.float32),
            pltpu.VMEM((R, D), jnp.float32),
            pltpu.VMEM((C + 16,), jnp.int32),
            pltpu.VMEM((D,), jnp.float32),
            pltpu.VMEM((16,), jnp.int32),
            pltpu.VMEM((ACC_W,), jnp.float32),
            pltpu.VMEM((ACC_W,), jnp.float32),
            pltpu.VMEM((16,), jnp.int32),
            pltpu.VMEM((16,), jnp.float32),
            pltpu.VMEM((D,), jnp.float32),
            pltpu.SemaphoreType.DMA,
            pltpu.SemaphoreType.DMA,
        ],
    )
    return kern(x, b32, query)


def _combine_body(gf_ref, acc_ref, part_ref, out_ref, acc2):
    acc2[...] = acc_ref[...]
    for w in range(NW):
        gw = gf_ref[w]
        acc2[pl.ds(gw, 1), :] = (acc2[pl.ds(gw, 1), :]
                                 + part_ref[pl.ds(w, 1), :])
    num = acc2[:, 0:128]
    den = acc2[:, 128:129]
    out_ref[...] = jnp.where(den > 0.0, num / den, 0.0)


def _combine(gfirst, acc, part):
    return pl.pallas_call(
        _combine_body,
        in_specs=[pl.BlockSpec(memory_space=pltpu.SMEM),
                  pl.BlockSpec((G, ACC_W), lambda: (0, 0)),
                  pl.BlockSpec((NW, ACC_W), lambda: (0, 0))],
        out_specs=pl.BlockSpec((G, D), lambda: (0, 0)),
        out_shape=jax.ShapeDtypeStruct((G, D), jnp.float32),
        scratch_shapes=[pltpu.VMEM((G, ACC_W), jnp.float32)],
    )(gfirst, acc, part)


def kernel(x, batch, query):
    b32 = batch.astype(jnp.int32)
    gfirst = b32[::C]
    out_flat, part_flat = _sc_pool(x, b32, query)
    return _combine(gfirst, out_flat.reshape(G, ACC_W),
                    part_flat.reshape(NW, ACC_W))


# speedup vs baseline: 1.1213x; 1.1213x over previous
import jax
import jax.numpy as jnp
from jax import lax
from jax.experimental import pallas as pl
from jax.experimental.pallas import tpu as pltpu
from jax.experimental.pallas import tpu_sc as plsc

N, D, G = 320000, 128, 1024
NCORES, NSUB = 2, 16
NW = NCORES * NSUB
C = N // NW
R = 400
NBLK = C // R
NGRP = R // 16
ACC_W = D + 16


def _sc_body(x_hbm, b_hbm, q_hbm, out_hbm, part_hbm,
             xb0, xb1, b_buf, q_buf, bp_buf, srow, zrow,
             g_st, den_st, acc_st, sem0, sem1):
    lane = lax.iota(jnp.int32, 16)
    zf = jnp.zeros((16,), jnp.float32)
    zi = jnp.zeros((16,), jnp.int32)
    cid = lax.axis_index("c")
    sid = lax.axis_index("s")
    wid = cid * NSUB + sid
    base = wid * C

    def _bcast(vec, idx16):
        dnums = lax.GatherDimensionNumbers(
            offset_dims=(), collapsed_slice_dims=(0,), start_index_map=(0,))
        return lax.gather(vec, idx16[:, None], dnums, (1,),
                          mode=lax.GatherScatterMode.PROMISE_IN_BOUNDS)

    def _allsum(v):
        for shift in (8, 4, 2, 1):
            v = v + _bcast(v, jnp.bitwise_xor(lane, shift))
        return v

    pltpu.sync_copy(b_hbm.at[pl.ds(base, C)], b_buf.at[pl.ds(0, C)])
    pltpu.sync_copy(q_hbm, q_buf)
    bp_off = pl.multiple_of(jnp.maximum(base - 16, 0), 8)
    pltpu.sync_copy(b_hbm.at[pl.ds(bp_off, 16)], bp_buf)

    g_prev = jnp.where(wid == 0, -1, bp_buf[...][15])
    g_first = b_buf[pl.ds(0, 16)][0]
    g_last = b_buf[pl.ds(C - 16, 16)][15]

    for k in range(9):
        zrow[pl.ds(16 * k, 16)] = zf

    def _zero_row(gid):
        pltpu.sync_copy(zrow, out_hbm.at[pl.ds(gid * ACC_W, ACC_W)])

    lax.fori_loop(0, g_first - g_prev,
                  lambda i, c: (_zero_row(g_prev + 1 + i), c)[1], 0)

    @pl.when(wid == NW - 1)
    def _():
        lax.fori_loop(0, G - 1 - g_last,
                      lambda i, c: (_zero_row(g_last + 1 + i), c)[1], 0)

    def _load_accs():
        return [acc_st[pl.ds(16 * k, 16)] for k in range(8)]

    def _emit(g_s, den_vec, accs):
        for k in range(8):
            srow[pl.ds(16 * k, 16)] = accs[k]
        srow[pl.ds(128, 16)] = den_vec

        @pl.when(g_s == g_first)
        def _():
            pltpu.sync_copy(srow, part_hbm.at[pl.ds(wid * ACC_W, ACC_W)])

        @pl.when(g_s != g_first)
        def _():
            pltpu.sync_copy(srow, out_hbm.at[pl.ds(g_s * ACC_W, ACC_W)])

    def _make_group_body(xb, pos0, xb_off, gsz):
        def body(grp, carry):
            s0 = xb_off + grp * gsz
            pos = pos0 + s0
            qk = [q_buf[pl.ds(16 * k, 16)] for k in range(8)]
            g = g_st[...][0]
            uniform = b_buf[pl.ds(pos + gsz - 1, 16)][0] == g

            def _row_e(row):
                xk = [xb[row, pl.ds(16 * k, 16)] for k in range(8)]
                p = xk[0] * qk[0]
                for k in range(1, 8):
                    p = p + xk[k] * qk[k]
                return xk, jnp.exp(_allsum(p))

            @pl.when(uniform)
            def _():
                accs = _load_accs()
                den = den_st[...]
                for r in range(gsz):
                    xk, e_bc = _row_e(s0 + r)
                    accs = [accs[k] + e_bc * xk[k] for k in range(8)]
                    den = den + e_bc
                for k in range(8):
                    acc_st[pl.ds(16 * k, 16)] = accs[k]
                den_st[...] = den

            @pl.when(jnp.logical_not(uniform))
            def _():
                def row_body(r, c):
                    b_r = b_buf[pl.ds(pos + r, 16)][0]
                    g_c = g_st[...][0]
                    pred = b_r != g_c

                    @pl.when(pred)
                    def _():
                        _emit(g_c, den_st[...], _load_accs())
                        lax.fori_loop(0, b_r - g_c - 1,
                                      lambda i, cc:
                                      (_zero_row(g_c + 1 + i), cc)[1], 0)
                        for k in range(8):
                            acc_st[pl.ds(16 * k, 16)] = zf
                        den_st[...] = zf
                        g_st[...] = zi + b_r

                    xk, e_bc = _row_e(s0 + r)
                    for k in range(8):
                        acc_st[pl.ds(16 * k, 16)] = (
                            acc_st[pl.ds(16 * k, 16)] + e_bc * xk[k])
                    den_st[...] = den_st[...] + e_bc
                    return c

                lax.fori_loop(0, gsz, row_body, 0)

            return carry
        return body

    def _process_block(xb, pos0, carry):
        return lax.fori_loop(0, NGRP, _make_group_body(xb, pos0, 0, 16),
                             carry)

    g_st[...] = zi + g_first
    den_st[...] = zf
    for k in range(8):
        acc_st[pl.ds(16 * k, 16)] = zf
    carry = 0

    def start(blk, buf, sem):
        pltpu.async_copy(x_hbm.at[pl.ds(base + blk * R, R)], buf, sem)

    def wait(blk, buf, sem):
        pltpu.make_async_copy(
            x_hbm.at[pl.ds(base + blk * R, R)], buf, sem).wait()

    start(0, xb0, sem0)

    def bb_body(bb, carry):
        b_even = 2 * bb
        start(b_even + 1, xb1, sem1)
        wait(b_even, xb0, sem0)
        carry = _process_block(xb0, b_even * R, carry)
        start(b_even + 2, xb0, sem0)
        wait(b_even + 1, xb1, sem1)
        carry = _process_block(xb1, (b_even + 1) * R, carry)
        return carry

    carry = lax.fori_loop(0, (NBLK - 1) // 2, bb_body, carry)
    wait(NBLK - 1, xb0, sem0)
    carry = _process_block(xb0, (NBLK - 1) * R, carry)
    _emit(g_st[...][0], den_st[...], _load_accs())


def _sc_pool(x, b32, query):
    mesh = plsc.VectorSubcoreMesh(core_axis_name="c", subcore_axis_name="s")
    kern = pl.kernel(
        _sc_body,
        mesh=mesh,
        out_type=[jax.ShapeDtypeStruct((G * ACC_W,), jnp.float32),
                  jax.ShapeDtypeStruct((NW * ACC_W,), jnp.float32)],
        scratch_types=[
            pltpu.VMEM((R, D), jnp.float32),
            pltpu.VMEM((R, D), jnp.float32),
            pltpu.VMEM((C + 16,), jnp.int32),
            pltpu.VMEM((D,), jnp.float32),
            pltpu.VMEM((16,), jnp.int32),
            pltpu.VMEM((ACC_W,), jnp.float32),
            pltpu.VMEM((ACC_W,), jnp.float32),
            pltpu.VMEM((16,), jnp.int32),
            pltpu.VMEM((16,), jnp.float32),
            pltpu.VMEM((D,), jnp.float32),
            pltpu.SemaphoreType.DMA,
            pltpu.SemaphoreType.DMA,
        ],
    )
    return kern(x, b32, query)


def _combine_body(gf_ref, acc_ref, part_ref, out_ref, acc2):
    acc2[...] = acc_ref[...]
    for w in range(NW):
        gw = gf_ref[w]
        acc2[pl.ds(gw, 1), :] = (acc2[pl.ds(gw, 1), :]
                                 + part_ref[pl.ds(w, 1), :])
    num = acc2[:, 0:128]
    den = acc2[:, 128:129]
    out_ref[...] = jnp.where(den > 0.0, num / den, 0.0)


def _combine(gfirst, acc, part):
    return pl.pallas_call(
        _combine_body,
        in_specs=[pl.BlockSpec(memory_space=pltpu.SMEM),
                  pl.BlockSpec((G, ACC_W), lambda: (0, 0)),
                  pl.BlockSpec((NW, ACC_W), lambda: (0, 0))],
        out_specs=pl.BlockSpec((G, D), lambda: (0, 0)),
        out_shape=jax.ShapeDtypeStruct((G, D), jnp.float32),
        scratch_shapes=[pltpu.VMEM((G, ACC_W), jnp.float32)],
    )(gfirst, acc, part)


def kernel(x, batch, query):
    b32 = batch.astype(jnp.int32)
    gfirst = b32[::C]
    out_flat, part_flat = _sc_pool(x, b32, query)
    return _combine(gfirst, out_flat.reshape(G, ACC_W),
                    part_flat.reshape(NW, ACC_W))
